# Initial kernel scaffold; baseline (speedup 1.0000x reference)
import jax
import jax.numpy as jnp
from jax.experimental import pallas as pl


def kernel(x, edge_index, Wp, bp, Wl1, bl1, Wr1, Wl2, bl2, Wr2):
    raise NotImplementedError("write your pallas kernel here")


# speedup vs baseline: 5.8639x; 5.8639x over previous
import functools

import jax
import jax.numpy as jnp
from jax import lax
from jax.experimental import pallas as pl
from jax.experimental.pallas import tpu as pltpu
from jax.experimental.pallas import tpu_sc as plsc

N = 10000
E = 320000
D = 128
H = 64
W2 = 16
NC = 2
NS = 16
NPAD = 10240
RPT = NPAD // NS
CH = 128

BLK = 400

_mesh = plsc.VectorSubcoreMesh(core_axis_name="c", subcore_axis_name="s")
_sc_params = pltpu.CompilerParams(use_tc_tiling_on_sc=False)


def _dotT(a, b):
    return lax.dot_general(a, b, (((1,), (1,)), ((), ())),
                           preferred_element_type=jnp.float32)


def _tc_pre_body(x_ref, wp_ref, bp_ref, wl1_ref, wr1_ref, bl1_ref,
                 m1_ref, r1_ref):
    h0 = jnp.maximum(_dotT(x_ref[...], wp_ref[...]) + bp_ref[...], 0.0)
    m1 = _dotT(h0, wl1_ref[...])
    m1_ref[0] = m1[:, :H]
    m1_ref[1] = m1[:, H:]
    r1_ref[...] = _dotT(h0, wr1_ref[...]) + bl1_ref[...]


_tc_pre = pl.pallas_call(
    _tc_pre_body,
    grid=(N // BLK,),
    in_specs=[
        pl.BlockSpec((BLK, D), lambda i: (i, 0)),
        pl.BlockSpec((D, D), lambda i: (0, 0)),
        pl.BlockSpec((1, D), lambda i: (0, 0)),
        pl.BlockSpec((D, D), lambda i: (0, 0)),
        pl.BlockSpec((D, D), lambda i: (0, 0)),
        pl.BlockSpec((1, D), lambda i: (0, 0)),
    ],
    out_specs=[
        pl.BlockSpec((NC, BLK, H), lambda i: (0, i, 0)),
        pl.BlockSpec((BLK, D), lambda i: (i, 0)),
    ],
    out_shape=[
        jax.ShapeDtypeStruct((NC, N, H), jnp.float32),
        jax.ShapeDtypeStruct((N, D), jnp.float32),
    ],
)


EPT1 = E // NS
NCH1 = EPT1 // CH
REM1 = EPT1 - NCH1 * CH


def _sc1_body(m1cat, src_e, dst_e, agg_out, deg_out,
              src_v, dst_v, rows_v, src_r, dst_r, rows_r,
              zb, zbd, ones_v, ones_r, acc_sh, deg_sh, sem):
    c = lax.axis_index("c")
    s = lax.axis_index("s")
    is0 = c == 0
    z16 = jnp.zeros((16,), jnp.float32)
    o16 = jnp.ones((16,), jnp.float32)

    @pl.loop(0, CH)
    def _fill(i):
        for j in range(H // 16):
            zb[i, pl.ds(16 * j, 16)] = z16
        zbd[i, :] = z16
        ones_v[i, :] = o16

    @pl.loop(0, REM1)
    def _fill_r(i):
        ones_r[i, :] = o16

    for t in range(RPT // CH):
        r0 = s * RPT + t * CH
        pltpu.sync_copy(zb, acc_sh.at[pl.ds(r0, CH), :])
        pltpu.sync_copy(zbd, deg_sh.at[pl.ds(r0, CH), :])
    plsc.subcore_barrier()

    ebase = s * EPT1
    roff = c * N

    @pl.loop(0, NCH1)
    def _chunk(k):
        base = ebase + k * CH
        pltpu.sync_copy(src_e.at[pl.ds(base, CH)], src_v)
        for j in range(CH // 16):
            sl = pl.ds(16 * j, 16)
            src_v[sl] = src_v[sl] + roff
        pltpu.async_copy(m1cat.at[src_v], rows_v, sem).wait()
        pltpu.sync_copy(dst_e.at[pl.ds(base, CH)], dst_v)
        pltpu.sync_copy(rows_v, acc_sh.at[dst_v], add=True)

        @pl.when(is0)
        def _deg():
            pltpu.sync_copy(ones_v, deg_sh.at[dst_v], add=True)

    base_r = ebase + NCH1 * CH
    pltpu.sync_copy(src_e.at[pl.ds(base_r, REM1)], src_r)
    for j in range(REM1 // 16):
        sl = pl.ds(16 * j, 16)
        src_r[sl] = src_r[sl] + roff
    pltpu.async_copy(m1cat.at[src_r], rows_r, sem).wait()
    pltpu.sync_copy(dst_e.at[pl.ds(base_r, REM1)], dst_r)
    pltpu.sync_copy(rows_r, acc_sh.at[dst_r], add=True)

    @pl.when(is0)
    def _deg_r():
        pltpu.sync_copy(ones_r, deg_sh.at[dst_r], add=True)

    plsc.subcore_barrier()

    @pl.when(s < NS - 1)
    def _out_full():
        r0 = s * RPT
        pltpu.sync_copy(acc_sh.at[pl.ds(r0, RPT), :],
                        agg_out.at[pl.ds(c * N + r0, RPT), :])

        @pl.when(is0)
        def _():
            pltpu.sync_copy(deg_sh.at[pl.ds(r0, RPT), :],
                            deg_out.at[pl.ds(r0, RPT), :])

    @pl.when(s == NS - 1)
    def _out_last():
        r0 = (NS - 1) * RPT
        nlast = N - r0
        pltpu.sync_copy(acc_sh.at[pl.ds(r0, nlast), :],
                        agg_out.at[pl.ds(c * N + r0, nlast), :])

        @pl.when(is0)
        def _():
            pltpu.sync_copy(deg_sh.at[pl.ds(r0, nlast), :],
                            deg_out.at[pl.ds(r0, nlast), :])


_sc_agg1 = pl.kernel(
    _sc1_body,
    out_type=(
        jax.ShapeDtypeStruct((NC * N, H), jnp.float32),
        jax.ShapeDtypeStruct((N, W2), jnp.float32),
    ),
    mesh=_mesh,
    scratch_types=(
        pltpu.VMEM((CH,), jnp.int32),
        pltpu.VMEM((CH,), jnp.int32),
        pltpu.VMEM((CH, H), jnp.float32),
        pltpu.VMEM((REM1,), jnp.int32),
        pltpu.VMEM((REM1,), jnp.int32),
        pltpu.VMEM((REM1, H), jnp.float32),
        pltpu.VMEM((CH, H), jnp.float32),
        pltpu.VMEM((CH, W2), jnp.float32),
        pltpu.VMEM((CH, W2), jnp.float32),
        pltpu.VMEM((REM1, W2), jnp.float32),
        pltpu.VMEM_SHARED((NPAD, H), jnp.float32),
        pltpu.VMEM_SHARED((NPAD, W2), jnp.float32),
        pltpu.SemaphoreType.DMA,
    ),
    compiler_params=_sc_params,
)


def _tc_mid_body(aggp_ref, deg_ref, r1_ref, wl2_ref, wr2_ref, bl2_ref,
                 m2_ref, aux_ref):
    agg = jnp.concatenate([aggp_ref[0], aggp_ref[1]], axis=1)
    degc = jnp.maximum(deg_ref[:, 0], 1.0)
    h1 = jnp.maximum(agg / degc[:, None] + r1_ref[...], 0.0)
    m2_ref[...] = _dotT(h1, wl2_ref[...])
    r2 = _dotT(h1, wr2_ref[...]) + bl2_ref[...]
    col = lax.broadcasted_iota(jnp.int32, (BLK, W2), 1)
    aux_ref[...] = jnp.where(col == 8, degc[:, None], r2)


_tc_mid = pl.pallas_call(
    _tc_mid_body,
    grid=(N // BLK,),
    in_specs=[
        pl.BlockSpec((NC, BLK, H), lambda i: (0, i, 0)),
        pl.BlockSpec((BLK, W2), lambda i: (i, 0)),
        pl.BlockSpec((BLK, D), lambda i: (i, 0)),
        pl.BlockSpec((W2, D), lambda i: (0, 0)),
        pl.BlockSpec((W2, D), lambda i: (0, 0)),
        pl.BlockSpec((1, W2), lambda i: (0, 0)),
    ],
    out_specs=[
        pl.BlockSpec((BLK, W2), lambda i: (i, 0)),
        pl.BlockSpec((BLK, W2), lambda i: (i, 0)),
    ],
    out_shape=[
        jax.ShapeDtypeStruct((N, W2), jnp.float32),
        jax.ShapeDtypeStruct((N, W2), jnp.float32),
    ],
)


EPT2 = E // (NC * NS)
NCH2 = EPT2 // CH
REM2 = EPT2 - NCH2 * CH


def _sc2_body(m2pad, src_e, dst_e, agg_out,
              src_v, dst_v, rows_v, src_r, dst_r, rows_r, zb, acc_sh, sem):
    c = lax.axis_index("c")
    s = lax.axis_index("s")
    z16 = jnp.zeros((16,), jnp.float32)

    @pl.loop(0, CH)
    def _fill(i):
        zb[i, :] = z16

    for t in range(RPT // CH):
        pltpu.sync_copy(zb, acc_sh.at[pl.ds(s * RPT + t * CH, CH), :])
    plsc.subcore_barrier()

    ebase = c * (E // NC) + s * EPT2

    @pl.loop(0, NCH2)
    def _chunk(k):
        base = ebase + k * CH
        pltpu.sync_copy(src_e.at[pl.ds(base, CH)], src_v)
        pltpu.async_copy(m2pad.at[src_v], rows_v, sem).wait()
        pltpu.sync_copy(dst_e.at[pl.ds(base, CH)], dst_v)
        pltpu.sync_copy(rows_v, acc_sh.at[dst_v], add=True)

    base_r = ebase + NCH2 * CH
    pltpu.sync_copy(src_e.at[pl.ds(base_r, REM2)], src_r)
    pltpu.async_copy(m2pad.at[src_r], rows_r, sem).wait()
    pltpu.sync_copy(dst_e.at[pl.ds(base_r, REM2)], dst_r)
    pltpu.sync_copy(rows_r, acc_sh.at[dst_r], add=True)

    plsc.subcore_barrier()

    @pl.when(s < NS - 1)
    def _out_full():
        r0 = s * RPT
        pltpu.sync_copy(acc_sh.at[pl.ds(r0, RPT), :],
                        agg_out.at[pl.ds(c * N + r0, RPT), :])

    @pl.when(s == NS - 1)
    def _out_last():
        r0 = (NS - 1) * RPT
        nlast = N - r0
        pltpu.sync_copy(acc_sh.at[pl.ds(r0, nlast), :],
                        agg_out.at[pl.ds(c * N + r0, nlast), :])


_sc_agg2 = pl.kernel(
    _sc2_body,
    out_type=jax.ShapeDtypeStruct((NC * N, W2), jnp.float32),
    mesh=_mesh,
    scratch_types=(
        pltpu.VMEM((CH,), jnp.int32),
        pltpu.VMEM((CH,), jnp.int32),
        pltpu.VMEM((CH, W2), jnp.float32),
        pltpu.VMEM((REM2,), jnp.int32),
        pltpu.VMEM((REM2,), jnp.int32),
        pltpu.VMEM((REM2, W2), jnp.float32),
        pltpu.VMEM((CH, W2), jnp.float32),
        pltpu.VMEM_SHARED((NPAD, W2), jnp.float32),
        pltpu.SemaphoreType.DMA,
    ),
    compiler_params=_sc_params,
)


def _tc_fin_body(agg2p_ref, aux_ref, out_ref):
    p = agg2p_ref[0] + agg2p_ref[1]
    out_ref[...] = p[:, 0:4] / aux_ref[:, 8:9] + aux_ref[:, 0:4]


_tc_fin = pl.pallas_call(
    _tc_fin_body,
    grid=(N // BLK,),
    in_specs=[
        pl.BlockSpec((NC, BLK, W2), lambda i: (0, i, 0)),
        pl.BlockSpec((BLK, W2), lambda i: (i, 0)),
    ],
    out_specs=pl.BlockSpec((BLK, 4), lambda i: (i, 0)),
    out_shape=jax.ShapeDtypeStruct((N, 4), jnp.float32),
)


def kernel(x, edge_index, Wp, bp, Wl1, bl1, Wr1, Wl2, bl2, Wr2):
    src = edge_index[0].astype(jnp.int32)
    dst = edge_index[1].astype(jnp.int32)
    m1_parts, r1 = _tc_pre(x, Wp, bp.reshape(1, D), Wl1, Wr1, bl1.reshape(1, D))
    aggf, degf = _sc_agg1(m1_parts.reshape(NC * N, H), src, dst)
    wl2p = jnp.zeros((W2, D), jnp.float32).at[:4].set(Wl2)
    wr2p = jnp.zeros((W2, D), jnp.float32).at[:4].set(Wr2)
    bl2p = jnp.zeros((1, W2), jnp.float32).at[0, :4].set(bl2)
    m2pad, aux = _tc_mid(aggf.reshape(NC, N, H), degf, r1, wl2p, wr2p, bl2p)
    agg2f = _sc_agg2(m2pad, src, dst)
    return _tc_fin(agg2f.reshape(NC, N, W2), aux)


# speedup vs baseline: 6.1783x; 1.0536x over previous
import functools

import jax
import jax.numpy as jnp
from jax import lax
from jax.experimental import pallas as pl
from jax.experimental.pallas import tpu as pltpu
from jax.experimental.pallas import tpu_sc as plsc

N = 10000
E = 320000
D = 128
H = 64
W2 = 16
NC = 2
NS = 16
NPAD = 10240
RPT = NPAD // NS
CH = 128

BLK = 400

_mesh = plsc.VectorSubcoreMesh(core_axis_name="c", subcore_axis_name="s")
_sc_params = pltpu.CompilerParams(use_tc_tiling_on_sc=False)


def _dotT(a, b):
    return lax.dot_general(a, b, (((1,), (1,)), ((), ())),
                           preferred_element_type=jnp.float32)


def _tc_pre_body(x_ref, wp_ref, bp_ref, wl1_ref, wr1_ref, bl1_ref,
                 m1_ref, r1_ref):
    h0 = jnp.maximum(_dotT(x_ref[...], wp_ref[...]) + bp_ref[...], 0.0)
    m1 = _dotT(h0, wl1_ref[...])
    m1_ref[0] = m1[:, :H]
    m1_ref[1] = m1[:, H:]
    r1_ref[...] = _dotT(h0, wr1_ref[...]) + bl1_ref[...]


_tc_pre = pl.pallas_call(
    _tc_pre_body,
    grid=(N // BLK,),
    in_specs=[
        pl.BlockSpec((BLK, D), lambda i: (i, 0)),
        pl.BlockSpec((D, D), lambda i: (0, 0)),
        pl.BlockSpec((1, D), lambda i: (0, 0)),
        pl.BlockSpec((D, D), lambda i: (0, 0)),
        pl.BlockSpec((D, D), lambda i: (0, 0)),
        pl.BlockSpec((1, D), lambda i: (0, 0)),
    ],
    out_specs=[
        pl.BlockSpec((NC, BLK, H), lambda i: (0, i, 0)),
        pl.BlockSpec((BLK, D), lambda i: (i, 0)),
    ],
    out_shape=[
        jax.ShapeDtypeStruct((NC, N, H), jnp.float32),
        jax.ShapeDtypeStruct((N, D), jnp.float32),
    ],
)


E2 = 327680
NCHK = E2 // CH
ROWS1 = NCHK // NS
ROWS2 = NCHK // (NC * NS)
NB = 4


def _sc1_body(m1cat, src_e, dst_e, agg_out, deg_out,
              src_all, dst_v, rows, zbd, ones_v,
              acc_sh, deg_sh, g0, g1, g2, g3):
    c = lax.axis_index("c")
    s = lax.axis_index("s")
    is0 = c == 0
    gsem = (g0, g1, g2, g3)
    z16 = jnp.zeros((16,), jnp.float32)
    o16 = jnp.ones((16,), jnp.float32)

    pltpu.sync_copy(src_e.at[pl.ds(s * ROWS1, ROWS1), :], src_all)
    roff = c * N

    @pl.loop(0, ROWS1)
    def _off(i):
        for j in range(CH // 16):
            sl = pl.ds(16 * j, 16)
            src_all[i, sl] = src_all[i, sl] + roff

    @pl.loop(0, CH)
    def _fill(i):
        for j in range(H // 16):
            rows[0, i, pl.ds(16 * j, 16)] = z16
        zbd[i, :] = z16
        ones_v[i, :] = o16

    for t in range(RPT // CH):
        r0 = s * RPT + t * CH
        pltpu.sync_copy(rows.at[0], acc_sh.at[pl.ds(r0, CH), :])
        pltpu.sync_copy(zbd, deg_sh.at[pl.ds(r0, CH), :])

    plsc.subcore_barrier()

    rbase = s * ROWS1

    @pl.loop(0, ROWS1 // NB)
    def _grp(q):
        c0 = q * NB
        descs = [pltpu.async_copy(m1cat.at[src_all.at[c0 + b]], rows.at[b],
                                  gsem[b]) for b in range(NB)]
        for b in range(NB):
            pltpu.sync_copy(dst_e.at[rbase + c0 + b], dst_v)
            descs[b].wait()
            pltpu.sync_copy(rows.at[b], acc_sh.at[dst_v], add=True)

            @pl.when(is0)
            def _deg():
                pltpu.sync_copy(ones_v, deg_sh.at[dst_v], add=True)

    plsc.subcore_barrier()

    @pl.when(s < NS - 1)
    def _out_full():
        r0 = s * RPT
        pltpu.sync_copy(acc_sh.at[pl.ds(r0, RPT), :],
                        agg_out.at[pl.ds(c * N + r0, RPT), :])

        @pl.when(is0)
        def _():
            pltpu.sync_copy(deg_sh.at[pl.ds(r0, RPT), :],
                            deg_out.at[pl.ds(r0, RPT), :])

    @pl.when(s == NS - 1)
    def _out_last():
        r0 = (NS - 1) * RPT
        nlast = N - r0
        pltpu.sync_copy(acc_sh.at[pl.ds(r0, nlast), :],
                        agg_out.at[pl.ds(c * N + r0, nlast), :])

        @pl.when(is0)
        def _():
            pltpu.sync_copy(deg_sh.at[pl.ds(r0, nlast), :],
                            deg_out.at[pl.ds(r0, nlast), :])


_sc_agg1 = pl.kernel(
    _sc1_body,
    out_type=(
        jax.ShapeDtypeStruct((NC * N, H), jnp.float32),
        jax.ShapeDtypeStruct((N, W2), jnp.float32),
    ),
    mesh=_mesh,
    scratch_types=(
        pltpu.VMEM((ROWS1, CH), jnp.int32),
        pltpu.VMEM((CH,), jnp.int32),
        pltpu.VMEM((NB, CH, H), jnp.float32),
        pltpu.VMEM((CH, W2), jnp.float32),
        pltpu.VMEM((CH, W2), jnp.float32),
        pltpu.VMEM_SHARED((NPAD, H), jnp.float32),
        pltpu.VMEM_SHARED((NPAD, W2), jnp.float32),
        pltpu.SemaphoreType.DMA,
        pltpu.SemaphoreType.DMA,
        pltpu.SemaphoreType.DMA,
        pltpu.SemaphoreType.DMA,
    ),
    compiler_params=_sc_params,
)


def _tc_mid_body(aggp_ref, deg_ref, r1_ref, wl2_ref, wr2_ref, bl2_ref,
                 m2_ref, aux_ref):
    agg = jnp.concatenate([aggp_ref[0], aggp_ref[1]], axis=1)
    degc = jnp.maximum(deg_ref[:, 0], 1.0)
    h1 = jnp.maximum(agg / degc[:, None] + r1_ref[...], 0.0)
    m2_ref[...] = _dotT(h1, wl2_ref[...])
    r2 = _dotT(h1, wr2_ref[...]) + bl2_ref[...]
    col = lax.broadcasted_iota(jnp.int32, (BLK, W2), 1)
    aux_ref[...] = jnp.where(col == 8, degc[:, None], r2)


_tc_mid = pl.pallas_call(
    _tc_mid_body,
    grid=(N // BLK,),
    in_specs=[
        pl.BlockSpec((NC, BLK, H), lambda i: (0, i, 0)),
        pl.BlockSpec((BLK, W2), lambda i: (i, 0)),
        pl.BlockSpec((BLK, D), lambda i: (i, 0)),
        pl.BlockSpec((W2, D), lambda i: (0, 0)),
        pl.BlockSpec((W2, D), lambda i: (0, 0)),
        pl.BlockSpec((1, W2), lambda i: (0, 0)),
    ],
    out_specs=[
        pl.BlockSpec((BLK, W2), lambda i: (i, 0)),
        pl.BlockSpec((BLK, W2), lambda i: (i, 0)),
    ],
    out_shape=[
        jax.ShapeDtypeStruct((N, W2), jnp.float32),
        jax.ShapeDtypeStruct((N, W2), jnp.float32),
    ],
)


def _sc2_body(m2pad, src_e, dst_e, agg_out,
              src_all, dst_v, rows, acc_sh, g0, g1, g2, g3):
    c = lax.axis_index("c")
    s = lax.axis_index("s")
    gsem = (g0, g1, g2, g3)
    z16 = jnp.zeros((16,), jnp.float32)

    rbase = c * (NCHK // NC) + s * ROWS2
    pltpu.sync_copy(src_e.at[pl.ds(rbase, ROWS2), :], src_all)

    @pl.loop(0, CH)
    def _fill(i):
        rows[0, i, :] = z16

    for t in range(RPT // CH):
        pltpu.sync_copy(rows.at[0], acc_sh.at[pl.ds(s * RPT + t * CH, CH), :])

    plsc.subcore_barrier()

    @pl.loop(0, ROWS2 // NB)
    def _grp(q):
        c0 = q * NB
        descs = [pltpu.async_copy(m2pad.at[src_all.at[c0 + b]], rows.at[b],
                                  gsem[b]) for b in range(NB)]
        for b in range(NB):
            pltpu.sync_copy(dst_e.at[rbase + c0 + b], dst_v)
            descs[b].wait()
            pltpu.sync_copy(rows.at[b], acc_sh.at[dst_v], add=True)

    plsc.subcore_barrier()

    @pl.when(s < NS - 1)
    def _out_full():
        r0 = s * RPT
        pltpu.sync_copy(acc_sh.at[pl.ds(r0, RPT), :],
                        agg_out.at[pl.ds(c * N + r0, RPT), :])

    @pl.when(s == NS - 1)
    def _out_last():
        r0 = (NS - 1) * RPT
        nlast = N - r0
        pltpu.sync_copy(acc_sh.at[pl.ds(r0, nlast), :],
                        agg_out.at[pl.ds(c * N + r0, nlast), :])


_sc_agg2 = pl.kernel(
    _sc2_body,
    out_type=jax.ShapeDtypeStruct((NC * N, W2), jnp.float32),
    mesh=_mesh,
    scratch_types=(
        pltpu.VMEM((ROWS2, CH), jnp.int32),
        pltpu.VMEM((CH,), jnp.int32),
        pltpu.VMEM((NB, CH, W2), jnp.float32),
        pltpu.VMEM_SHARED((NPAD, W2), jnp.float32),
        pltpu.SemaphoreType.DMA,
        pltpu.SemaphoreType.DMA,
        pltpu.SemaphoreType.DMA,
        pltpu.SemaphoreType.DMA,
    ),
    compiler_params=_sc_params,
)


def _tc_fin_body(agg2p_ref, aux_ref, out_ref):
    p = agg2p_ref[0] + agg2p_ref[1]
    out_ref[...] = p[:, 0:4] / aux_ref[:, 8:9] + aux_ref[:, 0:4]


_tc_fin = pl.pallas_call(
    _tc_fin_body,
    grid=(N // BLK,),
    in_specs=[
        pl.BlockSpec((NC, BLK, W2), lambda i: (0, i, 0)),
        pl.BlockSpec((BLK, W2), lambda i: (i, 0)),
    ],
    out_specs=pl.BlockSpec((BLK, 4), lambda i: (i, 0)),
    out_shape=jax.ShapeDtypeStruct((N, 4), jnp.float32),
)


def kernel(x, edge_index, Wp, bp, Wl1, bl1, Wr1, Wl2, bl2, Wr2):
    pad = E2 - E
    src = jnp.concatenate([edge_index[0].astype(jnp.int32),
                           jnp.zeros((pad,), jnp.int32)]).reshape(NCHK, CH)
    dst = jnp.concatenate([edge_index[1].astype(jnp.int32),
                           jnp.full((pad,), NPAD - 1, jnp.int32)]).reshape(NCHK, CH)
    m1_parts, r1 = _tc_pre(x, Wp, bp.reshape(1, D), Wl1, Wr1, bl1.reshape(1, D))
    aggf, degf = _sc_agg1(m1_parts.reshape(NC * N, H), src, dst)
    wl2p = jnp.zeros((W2, D), jnp.float32).at[:4].set(Wl2)
    wr2p = jnp.zeros((W2, D), jnp.float32).at[:4].set(Wr2)
    bl2p = jnp.zeros((1, W2), jnp.float32).at[0, :4].set(bl2)
    m2pad, aux = _tc_mid(aggf.reshape(NC, N, H), degf, r1, wl2p, wr2p, bl2p)
    agg2f = _sc_agg2(m2pad, src, dst)
    return _tc_fin(agg2f.reshape(NC, N, W2), aux)


# speedup vs baseline: 7.4760x; 1.2101x over previous
import functools

import jax
import jax.numpy as jnp
from jax import lax
from jax.experimental import pallas as pl
from jax.experimental.pallas import tpu as pltpu
from jax.experimental.pallas import tpu_sc as plsc

N = 10000
E = 320000
D = 128
H = 64
W2 = 16
NC = 2
NS = 16
NPAD = 10240
RPT = NPAD // NS
CH = 128

BLK = 400

_mesh = plsc.VectorSubcoreMesh(core_axis_name="c", subcore_axis_name="s")
_sc_params = pltpu.CompilerParams(use_tc_tiling_on_sc=False)


def _dotT(a, b):
    return lax.dot_general(a, b, (((1,), (1,)), ((), ())),
                           preferred_element_type=jnp.float32)


def _tc_pre_body(x_ref, wp_ref, bp_ref, wl1_ref, wr1_ref, bl1_ref,
                 m1_ref, r1_ref):
    h0 = jnp.maximum(_dotT(x_ref[...], wp_ref[...]) + bp_ref[...], 0.0)
    m1 = _dotT(h0, wl1_ref[...])
    m1_ref[0] = m1[:, :H]
    m1_ref[1] = m1[:, H:]
    r1_ref[...] = _dotT(h0, wr1_ref[...]) + bl1_ref[...]


_tc_pre = pl.pallas_call(
    _tc_pre_body,
    grid=(N // BLK,),
    in_specs=[
        pl.BlockSpec((BLK, D), lambda i: (i, 0)),
        pl.BlockSpec((D, D), lambda i: (0, 0)),
        pl.BlockSpec((1, D), lambda i: (0, 0)),
        pl.BlockSpec((D, D), lambda i: (0, 0)),
        pl.BlockSpec((D, D), lambda i: (0, 0)),
        pl.BlockSpec((1, D), lambda i: (0, 0)),
    ],
    out_specs=[
        pl.BlockSpec((NC, BLK, H), lambda i: (0, i, 0)),
        pl.BlockSpec((BLK, D), lambda i: (i, 0)),
    ],
    out_shape=[
        jax.ShapeDtypeStruct((NC, N, H), jnp.float32),
        jax.ShapeDtypeStruct((N, D), jnp.float32),
    ],
)


E2 = 327680
NCHK = E2 // CH
ROWS1 = NCHK // NS
ROWS2 = NCHK // (NC * NS)
NB = 4


def _sc1_body(m1cat, src_e, dst_e, agg_out, deg_out,
              src_all, d0, d1, d2, d3, rows, zbd, ones_v,
              acc_sh, deg_sh, g0, g1, g2, g3, l0, l1, l2, l3, ssem, dsem):
    c = lax.axis_index("c")
    s = lax.axis_index("s")
    is0 = c == 0
    gsem = (g0, g1, g2, g3)
    lsem = (l0, l1, l2, l3)
    dref = (d0, d1, d2, d3)
    z16 = jnp.zeros((16,), jnp.float32)
    o16 = jnp.ones((16,), jnp.float32)

    pltpu.sync_copy(src_e.at[pl.ds(s * ROWS1, ROWS1), :], src_all)
    roff = c * N

    @pl.loop(0, ROWS1)
    def _off(i):
        for j in range(CH // 16):
            sl = pl.ds(16 * j, 16)
            src_all[i, sl] = src_all[i, sl] + roff

    @pl.loop(0, CH)
    def _fill(i):
        for j in range(H // 16):
            rows[0, i, pl.ds(16 * j, 16)] = z16
        zbd[i, :] = z16
        ones_v[i, :] = o16

    for t in range(RPT // CH):
        r0 = s * RPT + t * CH
        pltpu.sync_copy(rows.at[0], acc_sh.at[pl.ds(r0, CH), :])
        pltpu.sync_copy(zbd, deg_sh.at[pl.ds(r0, CH), :])

    plsc.subcore_barrier()

    rbase = s * ROWS1

    @pl.loop(0, ROWS1 // NB)
    def _grp(q):
        c0 = q * NB
        gd, ld = [], []
        for b in range(NB):
            gd.append(pltpu.async_copy(m1cat.at[src_all.at[c0 + b]],
                                       rows.at[b], gsem[b]))
            ld.append(pltpu.async_copy(dst_e.at[rbase + c0 + b], dref[b],
                                       lsem[b]))
        sd, dd = [], []
        for b in range(NB):
            ld[b].wait()
            gd[b].wait()
            sd.append(pltpu.async_copy(rows.at[b], acc_sh.at[dref[b]], ssem,
                                       add=True))

            @pl.when(is0)
            def _deg():
                pltpu.async_copy(ones_v, deg_sh.at[dref[b]], dsem, add=True)

            dd.append(pltpu.make_async_copy(ones_v, deg_sh.at[dref[b]], dsem))
        for b in range(NB):
            sd[b].wait()

            @pl.when(is0)
            def _degw():
                dd[b].wait()

    plsc.subcore_barrier()

    @pl.when(s < NS - 1)
    def _out_full():
        r0 = s * RPT
        pltpu.sync_copy(acc_sh.at[pl.ds(r0, RPT), :],
                        agg_out.at[pl.ds(c * N + r0, RPT), :])

        @pl.when(is0)
        def _():
            pltpu.sync_copy(deg_sh.at[pl.ds(r0, RPT), :],
                            deg_out.at[pl.ds(r0, RPT), :])

    @pl.when(s == NS - 1)
    def _out_last():
        r0 = (NS - 1) * RPT
        nlast = N - r0
        pltpu.sync_copy(acc_sh.at[pl.ds(r0, nlast), :],
                        agg_out.at[pl.ds(c * N + r0, nlast), :])

        @pl.when(is0)
        def _():
            pltpu.sync_copy(deg_sh.at[pl.ds(r0, nlast), :],
                            deg_out.at[pl.ds(r0, nlast), :])


_sc_agg1 = pl.kernel(
    _sc1_body,
    out_type=(
        jax.ShapeDtypeStruct((NC * N, H), jnp.float32),
        jax.ShapeDtypeStruct((N, W2), jnp.float32),
    ),
    mesh=_mesh,
    scratch_types=(
        pltpu.VMEM((ROWS1, CH), jnp.int32),
        pltpu.VMEM((CH,), jnp.int32),
        pltpu.VMEM((CH,), jnp.int32),
        pltpu.VMEM((CH,), jnp.int32),
        pltpu.VMEM((CH,), jnp.int32),
        pltpu.VMEM((NB, CH, H), jnp.float32),
        pltpu.VMEM((CH, W2), jnp.float32),
        pltpu.VMEM((CH, W2), jnp.float32),
        pltpu.VMEM_SHARED((NPAD, H), jnp.float32),
        pltpu.VMEM_SHARED((NPAD, W2), jnp.float32),
    ) + (pltpu.SemaphoreType.DMA,) * 10,
    compiler_params=_sc_params,
)


def _tc_mid_body(aggp_ref, deg_ref, r1_ref, wl2_ref, wr2_ref, bl2_ref,
                 m2_ref, aux_ref):
    agg = jnp.concatenate([aggp_ref[0], aggp_ref[1]], axis=1)
    degc = jnp.maximum(deg_ref[:, 0], 1.0)
    h1 = jnp.maximum(agg / degc[:, None] + r1_ref[...], 0.0)
    m2_ref[...] = _dotT(h1, wl2_ref[...])
    r2 = _dotT(h1, wr2_ref[...]) + bl2_ref[...]
    col = lax.broadcasted_iota(jnp.int32, (BLK, W2), 1)
    aux_ref[...] = jnp.where(col == 8, degc[:, None], r2)


_tc_mid = pl.pallas_call(
    _tc_mid_body,
    grid=(N // BLK,),
    in_specs=[
        pl.BlockSpec((NC, BLK, H), lambda i: (0, i, 0)),
        pl.BlockSpec((BLK, W2), lambda i: (i, 0)),
        pl.BlockSpec((BLK, D), lambda i: (i, 0)),
        pl.BlockSpec((W2, D), lambda i: (0, 0)),
        pl.BlockSpec((W2, D), lambda i: (0, 0)),
        pl.BlockSpec((1, W2), lambda i: (0, 0)),
    ],
    out_specs=[
        pl.BlockSpec((BLK, W2), lambda i: (i, 0)),
        pl.BlockSpec((BLK, W2), lambda i: (i, 0)),
    ],
    out_shape=[
        jax.ShapeDtypeStruct((N, W2), jnp.float32),
        jax.ShapeDtypeStruct((N, W2), jnp.float32),
    ],
)


def _sc2_body(m2pad, src_e, dst_e, agg_out,
              src_all, d0, d1, d2, d3, rows, acc_sh,
              g0, g1, g2, g3, l0, l1, l2, l3, ssem):
    c = lax.axis_index("c")
    s = lax.axis_index("s")
    gsem = (g0, g1, g2, g3)
    lsem = (l0, l1, l2, l3)
    dref = (d0, d1, d2, d3)
    z16 = jnp.zeros((16,), jnp.float32)

    rbase = c * (NCHK // NC) + s * ROWS2
    pltpu.sync_copy(src_e.at[pl.ds(rbase, ROWS2), :], src_all)

    @pl.loop(0, CH)
    def _fill(i):
        rows[0, i, :] = z16

    for t in range(RPT // CH):
        pltpu.sync_copy(rows.at[0], acc_sh.at[pl.ds(s * RPT + t * CH, CH), :])

    plsc.subcore_barrier()

    @pl.loop(0, ROWS2 // NB)
    def _grp(q):
        c0 = q * NB
        gd, ld = [], []
        for b in range(NB):
            gd.append(pltpu.async_copy(m2pad.at[src_all.at[c0 + b]],
                                       rows.at[b], gsem[b]))
            ld.append(pltpu.async_copy(dst_e.at[rbase + c0 + b], dref[b],
                                       lsem[b]))
        sd = []
        for b in range(NB):
            ld[b].wait()
            gd[b].wait()
            sd.append(pltpu.async_copy(rows.at[b], acc_sh.at[dref[b]], ssem,
                                       add=True))
        for b in range(NB):
            sd[b].wait()

    plsc.subcore_barrier()

    @pl.when(s < NS - 1)
    def _out_full():
        r0 = s * RPT
        pltpu.sync_copy(acc_sh.at[pl.ds(r0, RPT), :],
                        agg_out.at[pl.ds(c * N + r0, RPT), :])

    @pl.when(s == NS - 1)
    def _out_last():
        r0 = (NS - 1) * RPT
        nlast = N - r0
        pltpu.sync_copy(acc_sh.at[pl.ds(r0, nlast), :],
                        agg_out.at[pl.ds(c * N + r0, nlast), :])


_sc_agg2 = pl.kernel(
    _sc2_body,
    out_type=jax.ShapeDtypeStruct((NC * N, W2), jnp.float32),
    mesh=_mesh,
    scratch_types=(
        pltpu.VMEM((ROWS2, CH), jnp.int32),
        pltpu.VMEM((CH,), jnp.int32),
        pltpu.VMEM((CH,), jnp.int32),
        pltpu.VMEM((CH,), jnp.int32),
        pltpu.VMEM((CH,), jnp.int32),
        pltpu.VMEM((NB, CH, W2), jnp.float32),
        pltpu.VMEM_SHARED((NPAD, W2), jnp.float32),
    ) + (pltpu.SemaphoreType.DMA,) * 9,
    compiler_params=_sc_params,
)


def _tc_fin_body(agg2p_ref, aux_ref, out_ref):
    p = agg2p_ref[0] + agg2p_ref[1]
    out_ref[...] = p[:, 0:4] / aux_ref[:, 8:9] + aux_ref[:, 0:4]


_tc_fin = pl.pallas_call(
    _tc_fin_body,
    grid=(N // BLK,),
    in_specs=[
        pl.BlockSpec((NC, BLK, W2), lambda i: (0, i, 0)),
        pl.BlockSpec((BLK, W2), lambda i: (i, 0)),
    ],
    out_specs=pl.BlockSpec((BLK, 4), lambda i: (i, 0)),
    out_shape=jax.ShapeDtypeStruct((N, 4), jnp.float32),
)


def kernel(x, edge_index, Wp, bp, Wl1, bl1, Wr1, Wl2, bl2, Wr2):
    pad = E2 - E
    src = jnp.concatenate([edge_index[0].astype(jnp.int32),
                           jnp.zeros((pad,), jnp.int32)]).reshape(NCHK, CH)
    dst = jnp.concatenate([edge_index[1].astype(jnp.int32),
                           jnp.full((pad,), NPAD - 1, jnp.int32)]).reshape(NCHK, CH)
    m1_parts, r1 = _tc_pre(x, Wp, bp.reshape(1, D), Wl1, Wr1, bl1.reshape(1, D))
    aggf, degf = _sc_agg1(m1_parts.reshape(NC * N, H), src, dst)
    wl2p = jnp.zeros((W2, D), jnp.float32).at[:4].set(Wl2)
    wr2p = jnp.zeros((W2, D), jnp.float32).at[:4].set(Wr2)
    bl2p = jnp.zeros((1, W2), jnp.float32).at[0, :4].set(bl2)
    m2pad, aux = _tc_mid(aggf.reshape(NC, N, H), degf, r1, wl2p, wr2p, bl2p)
    agg2f = _sc_agg2(m2pad, src, dst)
    return _tc_fin(agg2f.reshape(NC, N, W2), aux)


# speedup vs baseline: 7.7416x; 1.0355x over previous
import functools

import jax
import jax.numpy as jnp
from jax import lax
from jax.experimental import pallas as pl
from jax.experimental.pallas import tpu as pltpu
from jax.experimental.pallas import tpu_sc as plsc

N = 10000
E = 320000
D = 128
H = 64
W2 = 16
NC = 2
NS = 16
NPAD = 10240
RPT = NPAD // NS
CH = 128

BLK = 400

_mesh = plsc.VectorSubcoreMesh(core_axis_name="c", subcore_axis_name="s")
_sc_params = pltpu.CompilerParams(use_tc_tiling_on_sc=False)


def _dotT(a, b):
    return lax.dot_general(a, b, (((1,), (1,)), ((), ())),
                           preferred_element_type=jnp.float32)


def _tc_pre_body(x_ref, wp_ref, bp_ref, wl1_ref, wr1_ref, bl1_ref,
                 m1_ref, r1_ref):
    h0 = jnp.maximum(_dotT(x_ref[...], wp_ref[...]) + bp_ref[...], 0.0)
    m1 = _dotT(h0, wl1_ref[...])
    m1_ref[0] = m1[:, :H]
    m1_ref[1] = m1[:, H:]
    r1_ref[...] = _dotT(h0, wr1_ref[...]) + bl1_ref[...]


_tc_pre = pl.pallas_call(
    _tc_pre_body,
    grid=(N // BLK,),
    in_specs=[
        pl.BlockSpec((BLK, D), lambda i: (i, 0)),
        pl.BlockSpec((D, D), lambda i: (0, 0)),
        pl.BlockSpec((1, D), lambda i: (0, 0)),
        pl.BlockSpec((D, D), lambda i: (0, 0)),
        pl.BlockSpec((D, D), lambda i: (0, 0)),
        pl.BlockSpec((1, D), lambda i: (0, 0)),
    ],
    out_specs=[
        pl.BlockSpec((NC, BLK, H), lambda i: (0, i, 0)),
        pl.BlockSpec((BLK, D), lambda i: (i, 0)),
    ],
    out_shape=[
        jax.ShapeDtypeStruct((NC, N, H), jnp.float32),
        jax.ShapeDtypeStruct((N, D), jnp.float32),
    ],
)


E2 = 327680
NCHK = E2 // CH
ROWS1 = NCHK // NS
ROWS2 = NCHK // (NC * NS)
NB = 5


def _sc1_body(m1cat, src_e, dst_e, agg_out, deg_out,
              src_all, d0, d1, d2, d3, d4, rows, zbd, ones_v,
              acc_sh, deg_sh, g0, g1, g2, g3, g4,
              l0, l1, l2, l3, l4, ssem, dsem):
    c = lax.axis_index("c")
    s = lax.axis_index("s")
    is0 = c == 0
    gsem = (g0, g1, g2, g3, g4)
    lsem = (l0, l1, l2, l3, l4)
    dref = (d0, d1, d2, d3, d4)
    z16 = jnp.zeros((16,), jnp.float32)
    o16 = jnp.ones((16,), jnp.float32)

    pltpu.sync_copy(src_e.at[pl.ds(s * ROWS1, ROWS1), :], src_all)
    roff = c * N

    @pl.loop(0, ROWS1)
    def _off(i):
        for j in range(CH // 16):
            sl = pl.ds(16 * j, 16)
            src_all[i, sl] = src_all[i, sl] + roff

    @pl.loop(0, CH)
    def _fill(i):
        for j in range(H // 16):
            rows[0, i, pl.ds(16 * j, 16)] = z16
        zbd[i, :] = z16
        ones_v[i, :] = o16

    for t in range(RPT // CH):
        r0 = s * RPT + t * CH
        pltpu.sync_copy(rows.at[0], acc_sh.at[pl.ds(r0, CH), :])
        pltpu.sync_copy(zbd, deg_sh.at[pl.ds(r0, CH), :])

    plsc.subcore_barrier()

    rbase = s * ROWS1

    @pl.loop(0, ROWS1 // NB)
    def _grp(q):
        c0 = q * NB
        gd, ld = [], []
        for b in range(NB):
            t = c0 + b
            gd.append(pltpu.async_copy(
                m1cat.at[src_all.at[t]], rows.at[b], gsem[b]))
            ld.append(pltpu.async_copy(dst_e.at[rbase + t], dref[b],
                                       lsem[b]))
        sd, dd = [], []
        for b in range(NB):
            ld[b].wait()
            gd[b].wait()
            sd.append(pltpu.async_copy(rows.at[b], acc_sh.at[dref[b]], ssem,
                                       add=True))

            @pl.when(is0)
            def _deg():
                pltpu.async_copy(ones_v, deg_sh.at[dref[b]], dsem, add=True)

            dd.append(pltpu.make_async_copy(ones_v, deg_sh.at[dref[b]], dsem))
        for b in range(NB):
            sd[b].wait()

            @pl.when(is0)
            def _degw():
                dd[b].wait()

    plsc.subcore_barrier()

    @pl.when(s < NS - 1)
    def _out_full():
        r0 = s * RPT
        pltpu.sync_copy(acc_sh.at[pl.ds(r0, RPT), :],
                        agg_out.at[pl.ds(c * N + r0, RPT), :])

        @pl.when(is0)
        def _():
            pltpu.sync_copy(deg_sh.at[pl.ds(r0, RPT), :],
                            deg_out.at[pl.ds(r0, RPT), :])

    @pl.when(s == NS - 1)
    def _out_last():
        r0 = (NS - 1) * RPT
        nlast = N - r0
        pltpu.sync_copy(acc_sh.at[pl.ds(r0, nlast), :],
                        agg_out.at[pl.ds(c * N + r0, nlast), :])

        @pl.when(is0)
        def _():
            pltpu.sync_copy(deg_sh.at[pl.ds(r0, nlast), :],
                            deg_out.at[pl.ds(r0, nlast), :])


_sc_agg1 = pl.kernel(
    _sc1_body,
    out_type=(
        jax.ShapeDtypeStruct((NC * N, H), jnp.float32),
        jax.ShapeDtypeStruct((N, W2), jnp.float32),
    ),
    mesh=_mesh,
    scratch_types=(
        pltpu.VMEM((ROWS1, CH), jnp.int32),
        pltpu.VMEM((CH,), jnp.int32),
        pltpu.VMEM((CH,), jnp.int32),
        pltpu.VMEM((CH,), jnp.int32),
        pltpu.VMEM((CH,), jnp.int32),
        pltpu.VMEM((CH,), jnp.int32),
        pltpu.VMEM((NB, CH, H), jnp.float32),
        pltpu.VMEM((CH, W2), jnp.float32),
        pltpu.VMEM((CH, W2), jnp.float32),
        pltpu.VMEM_SHARED((NPAD, H), jnp.float32),
        pltpu.VMEM_SHARED((NPAD, W2), jnp.float32),
    ) + (pltpu.SemaphoreType.DMA,) * 12,
    compiler_params=_sc_params,
)


def _tc_mid_body(aggp_ref, deg_ref, r1_ref, wl2_ref, wr2_ref, bl2_ref,
                 m2_ref, aux_ref):
    agg = jnp.concatenate([aggp_ref[0], aggp_ref[1]], axis=1)
    degc = jnp.maximum(deg_ref[:, 0], 1.0)
    h1 = jnp.maximum(agg / degc[:, None] + r1_ref[...], 0.0)
    m2_ref[...] = _dotT(h1, wl2_ref[...])
    r2 = _dotT(h1, wr2_ref[...]) + bl2_ref[...]
    col = lax.broadcasted_iota(jnp.int32, (BLK, W2), 1)
    aux_ref[...] = jnp.where(col == 8, degc[:, None], r2)


_tc_mid = pl.pallas_call(
    _tc_mid_body,
    grid=(N // BLK,),
    in_specs=[
        pl.BlockSpec((NC, BLK, H), lambda i: (0, i, 0)),
        pl.BlockSpec((BLK, W2), lambda i: (i, 0)),
        pl.BlockSpec((BLK, D), lambda i: (i, 0)),
        pl.BlockSpec((W2, D), lambda i: (0, 0)),
        pl.BlockSpec((W2, D), lambda i: (0, 0)),
        pl.BlockSpec((1, W2), lambda i: (0, 0)),
    ],
    out_specs=[
        pl.BlockSpec((BLK, W2), lambda i: (i, 0)),
        pl.BlockSpec((BLK, W2), lambda i: (i, 0)),
    ],
    out_shape=[
        jax.ShapeDtypeStruct((N, W2), jnp.float32),
        jax.ShapeDtypeStruct((N, W2), jnp.float32),
    ],
)


def _sc2_body(m2pad, src_e, dst_e, agg_out,
              src_all, d0, d1, d2, d3, d4, rows, acc_sh,
              g0, g1, g2, g3, g4, l0, l1, l2, l3, l4, ssem):
    c = lax.axis_index("c")
    s = lax.axis_index("s")
    gsem = (g0, g1, g2, g3, g4)
    lsem = (l0, l1, l2, l3, l4)
    dref = (d0, d1, d2, d3, d4)
    z16 = jnp.zeros((16,), jnp.float32)

    rbase = c * (NCHK // NC) + s * ROWS2
    pltpu.sync_copy(src_e.at[pl.ds(rbase, ROWS2), :], src_all)

    @pl.loop(0, CH)
    def _fill(i):
        rows[0, i, :] = z16

    for t in range(RPT // CH):
        pltpu.sync_copy(rows.at[0], acc_sh.at[pl.ds(s * RPT + t * CH, CH), :])

    plsc.subcore_barrier()

    @pl.loop(0, ROWS2 // NB)
    def _grp(q):
        c0 = q * NB
        gd, ld = [], []
        for b in range(NB):
            t = c0 + b
            gd.append(pltpu.async_copy(
                m2pad.at[src_all.at[t]], rows.at[b], gsem[b]))
            ld.append(pltpu.async_copy(dst_e.at[rbase + t], dref[b],
                                       lsem[b]))
        sd = []
        for b in range(NB):
            ld[b].wait()
            gd[b].wait()
            sd.append(pltpu.async_copy(rows.at[b], acc_sh.at[dref[b]], ssem,
                                       add=True))
        for b in range(NB):
            sd[b].wait()

    plsc.subcore_barrier()

    @pl.when(s < NS - 1)
    def _out_full():
        r0 = s * RPT
        pltpu.sync_copy(acc_sh.at[pl.ds(r0, RPT), :],
                        agg_out.at[pl.ds(c * N + r0, RPT), :])

    @pl.when(s == NS - 1)
    def _out_last():
        r0 = (NS - 1) * RPT
        nlast = N - r0
        pltpu.sync_copy(acc_sh.at[pl.ds(r0, nlast), :],
                        agg_out.at[pl.ds(c * N + r0, nlast), :])


_sc_agg2 = pl.kernel(
    _sc2_body,
    out_type=jax.ShapeDtypeStruct((NC * N, W2), jnp.float32),
    mesh=_mesh,
    scratch_types=(
        pltpu.VMEM((ROWS2, CH), jnp.int32),
        pltpu.VMEM((CH,), jnp.int32),
        pltpu.VMEM((CH,), jnp.int32),
        pltpu.VMEM((CH,), jnp.int32),
        pltpu.VMEM((CH,), jnp.int32),
        pltpu.VMEM((CH,), jnp.int32),
        pltpu.VMEM((NB, CH, W2), jnp.float32),
        pltpu.VMEM_SHARED((NPAD, W2), jnp.float32),
    ) + (pltpu.SemaphoreType.DMA,) * 11,
    compiler_params=_sc_params,
)


def _tc_fin_body(agg2p_ref, aux_ref, out_ref):
    p = agg2p_ref[0] + agg2p_ref[1]
    out_ref[...] = p[:, 0:4] / aux_ref[:, 8:9] + aux_ref[:, 0:4]


_tc_fin = pl.pallas_call(
    _tc_fin_body,
    grid=(N // BLK,),
    in_specs=[
        pl.BlockSpec((NC, BLK, W2), lambda i: (0, i, 0)),
        pl.BlockSpec((BLK, W2), lambda i: (i, 0)),
    ],
    out_specs=pl.BlockSpec((BLK, 4), lambda i: (i, 0)),
    out_shape=jax.ShapeDtypeStruct((N, 4), jnp.float32),
)


def kernel(x, edge_index, Wp, bp, Wl1, bl1, Wr1, Wl2, bl2, Wr2):
    pad = E2 - E
    src = jnp.concatenate([edge_index[0].astype(jnp.int32),
                           jnp.zeros((pad,), jnp.int32)]).reshape(NCHK, CH)
    dst = jnp.concatenate([edge_index[1].astype(jnp.int32),
                           jnp.full((pad,), NPAD - 1, jnp.int32)]).reshape(NCHK, CH)
    m1_parts, r1 = _tc_pre(x, Wp, bp.reshape(1, D), Wl1, Wr1, bl1.reshape(1, D))
    aggf, degf = _sc_agg1(m1_parts.reshape(NC * N, H), src, dst)
    wl2p = jnp.zeros((W2, D), jnp.float32).at[:4].set(Wl2)
    wr2p = jnp.zeros((W2, D), jnp.float32).at[:4].set(Wr2)
    bl2p = jnp.zeros((1, W2), jnp.float32).at[0, :4].set(bl2)
    m2pad, aux = _tc_mid(aggf.reshape(NC, N, H), degf, r1, wl2p, wr2p, bl2p)
    agg2f = _sc_agg2(m2pad, src, dst)
    return _tc_fin(agg2f.reshape(NC, N, W2), aux)
